# Initial kernel scaffold; baseline (speedup 1.0000x reference)
import functools

import jax
import jax.numpy as jnp
from jax import lax
from jax.experimental import pallas as pl
from jax.experimental.pallas import tpu as pltpu
from jax.experimental.pallas import tpu_sc as plsc

B = 4096
L = 200
D = 32
NC = 2
NS = 16
NW = NC * NS
BPW = B // NW
CB = 8
NCHUNK = BPW // CB
G = 80
NG = (CB * L) // G


def _sc_bag_sums(x_flat, table):
    mesh = plsc.VectorSubcoreMesh(
        core_axis_name="c", subcore_axis_name="s", num_cores=NC, num_subcores=NS
    )

    @functools.partial(
        pl.kernel,
        out_type=jax.ShapeDtypeStruct((B, D), jnp.float32),
        mesh=mesh,
        scratch_types=[
            pltpu.VMEM((CB * L,), jnp.int32),
            pltpu.VMEM((CB * L, D), jnp.float32),
            pltpu.VMEM((CB, D), jnp.float32),
            pltpu.SemaphoreType.DMA,
        ],
    )
    def k(x_hbm, tab_hbm, out_hbm, idx_v, rows_v, sum_v, sem):
        wid = lax.axis_index("s") * NC + lax.axis_index("c")
        wbase = wid * BPW

        @pl.loop(0, NCHUNK)
        def _chunk(g):
            bag0 = wbase + g * CB
            pltpu.sync_copy(x_hbm.at[pl.ds(bag0 * L, CB * L)], idx_v)
            descs = [
                pltpu.async_copy(
                    tab_hbm.at[idx_v.at[pl.ds(j * G, G)]],
                    rows_v.at[pl.ds(j * G, G), :],
                    sem,
                )
                for j in range(NG)
            ]
            for d in descs:
                d.wait()
            for bb in range(CB):
                base = bb * L
                zz = jnp.zeros((16,), jnp.float32)

                def body(i, carry, base=base):
                    a0, a1, a2, a3 = carry
                    r = base + 2 * i
                    a0 = a0 + rows_v[r, pl.ds(0, 16)]
                    a1 = a1 + rows_v[r, pl.ds(16, 16)]
                    a2 = a2 + rows_v[r + 1, pl.ds(0, 16)]
                    a3 = a3 + rows_v[r + 1, pl.ds(16, 16)]
                    return (a0, a1, a2, a3)

                a0, a1, a2, a3 = lax.fori_loop(0, L // 2, body, (zz, zz, zz, zz))
                sum_v[bb, pl.ds(0, 16)] = a0 + a2
                sum_v[bb, pl.ds(16, 16)] = a1 + a3
            pltpu.sync_copy(sum_v, out_hbm.at[pl.ds(bag0, CB)])

    return k(x_flat, table)


def _tc_head(x, summed, wt, b2):
    C = wt.shape[1]
    BT = 512

    def body(x_ref, s_ref, w_ref, b_ref, o_ref):
        cnt = jnp.sum((x_ref[...] != 0).astype(jnp.float32), axis=1, keepdims=True)
        denom = jnp.maximum(cnt, 1.0)
        acc = lax.dot_general(
            s_ref[...], w_ref[...], (((1,), (0,)), ((), ())),
            preferred_element_type=jnp.float32,
        )
        o_ref[...] = acc / denom + b_ref[...]

    return pl.pallas_call(
        body,
        grid=(B // BT,),
        in_specs=[
            pl.BlockSpec((BT, L), lambda i: (i, 0)),
            pl.BlockSpec((BT, D), lambda i: (i, 0)),
            pl.BlockSpec((D, C), lambda i: (0, 0)),
            pl.BlockSpec((1, C), lambda i: (0, 0)),
        ],
        out_specs=pl.BlockSpec((BT, C), lambda i: (i, 0)),
        out_shape=jax.ShapeDtypeStruct((B, C), jnp.float32),
    )(x, summed, wt, b2)


def kernel(x, table, W, b):
    x = x.astype(jnp.int32)
    summed = _sc_bag_sums(x.reshape(-1), table)
    return _tc_head(x, summed, W.T, b.reshape(1, -1))


# speedup vs baseline: 2.1979x; 2.1979x over previous
import functools

import jax
import jax.numpy as jnp
from jax import lax
from jax.experimental import pallas as pl
from jax.experimental.pallas import tpu as pltpu
from jax.experimental.pallas import tpu_sc as plsc

B = 4096
L = 200
D = 32
NC = 2
NS = 16
NW = NC * NS
BPW = B // NW
CB = 8
NCHUNK = BPW // CB
G = 80
NG = (CB * L) // G


def _sc_bag_sums(x_flat, table):
    mesh = plsc.VectorSubcoreMesh(
        core_axis_name="c", subcore_axis_name="s", num_cores=NC, num_subcores=NS
    )

    @functools.partial(
        pl.kernel,
        out_type=jax.ShapeDtypeStruct((B, D), jnp.float32),
        mesh=mesh,
        compiler_params=pltpu.CompilerParams(use_tc_tiling_on_sc=False),
        scratch_types=[
            pltpu.VMEM((CB * L,), jnp.int32),
            pltpu.VMEM((CB * L, D), jnp.float32),
            pltpu.VMEM((CB, D), jnp.float32),
            pltpu.SemaphoreType.DMA,
        ],
    )
    def k(x_hbm, tab_hbm, out_hbm, idx_v, rows_v, sum_v, sem):
        wid = lax.axis_index("s") * NC + lax.axis_index("c")
        wbase = wid * BPW

        @pl.loop(0, NCHUNK)
        def _chunk(g):
            bag0 = wbase + g * CB
            pltpu.sync_copy(x_hbm.at[pl.ds(bag0 * L, CB * L)], idx_v)
            descs = [
                pltpu.async_copy(
                    tab_hbm.at[idx_v.at[pl.ds(j * G, G)]],
                    rows_v.at[pl.ds(j * G, G), :],
                    sem,
                )
                for j in range(NG)
            ]
            for d in descs:
                d.wait()
            for bb in range(CB):
                base = bb * L
                zz = jnp.zeros((16,), jnp.float32)

                def body(i, carry, base=base):
                    a0, a1, a2, a3 = carry
                    r = base + 2 * i
                    a0 = a0 + rows_v[r, pl.ds(0, 16)]
                    a1 = a1 + rows_v[r, pl.ds(16, 16)]
                    a2 = a2 + rows_v[r + 1, pl.ds(0, 16)]
                    a3 = a3 + rows_v[r + 1, pl.ds(16, 16)]
                    return (a0, a1, a2, a3)

                a0, a1, a2, a3 = lax.fori_loop(0, L // 2, body, (zz, zz, zz, zz))
                sum_v[bb, pl.ds(0, 16)] = a0 + a2
                sum_v[bb, pl.ds(16, 16)] = a1 + a3
            pltpu.sync_copy(sum_v, out_hbm.at[pl.ds(bag0, CB)])

    return k(x_flat, table)


def _tc_head(x, summed, wt, b2):
    C = wt.shape[1]
    BT = 512

    def body(x_ref, s_ref, w_ref, b_ref, o_ref):
        cnt = jnp.sum((x_ref[...] != 0).astype(jnp.float32), axis=1, keepdims=True)
        denom = jnp.maximum(cnt, 1.0)
        acc = lax.dot_general(
            s_ref[...], w_ref[...], (((1,), (0,)), ((), ())),
            preferred_element_type=jnp.float32,
        )
        o_ref[...] = acc / denom + b_ref[...]

    return pl.pallas_call(
        body,
        grid=(B // BT,),
        in_specs=[
            pl.BlockSpec((BT, L), lambda i: (i, 0)),
            pl.BlockSpec((BT, D), lambda i: (i, 0)),
            pl.BlockSpec((D, C), lambda i: (0, 0)),
            pl.BlockSpec((1, C), lambda i: (0, 0)),
        ],
        out_specs=pl.BlockSpec((BT, C), lambda i: (i, 0)),
        out_shape=jax.ShapeDtypeStruct((B, C), jnp.float32),
    )(x, summed, wt, b2)


def kernel(x, table, W, b):
    x = x.astype(jnp.int32)
    summed = _sc_bag_sums(x.reshape(-1), table)
    return _tc_head(x, summed, W.T, b.reshape(1, -1))


# speedup vs baseline: 2.3957x; 1.0900x over previous
import functools

import jax
import jax.numpy as jnp
from jax import lax
from jax.experimental import pallas as pl
from jax.experimental.pallas import tpu as pltpu
from jax.experimental.pallas import tpu_sc as plsc

B = 4096
L = 200
D = 32
NC = 2
NS = 16
NW = NC * NS
BPW = B // NW
CB = 4
NCHUNK = BPW // CB
G0 = 104
G1 = L - G0


def _sc_bag_sums(x, table):
    mesh = plsc.VectorSubcoreMesh(
        core_axis_name="c", subcore_axis_name="s", num_cores=NC, num_subcores=NS
    )

    @functools.partial(
        pl.kernel,
        out_type=jax.ShapeDtypeStruct((B, D), jnp.float32),
        mesh=mesh,
        compiler_params=pltpu.CompilerParams(use_tc_tiling_on_sc=False),
        scratch_types=[
            pltpu.VMEM((BPW, L), jnp.int32),
            pltpu.VMEM((CB * L, D), jnp.float32),
            pltpu.VMEM((CB * L, D), jnp.float32),
            pltpu.VMEM((BPW, D), jnp.float32),
            pltpu.SemaphoreType.DMA,
            pltpu.SemaphoreType.DMA,
        ],
    )
    def k(x_hbm, tab_hbm, out_hbm, idx_all, rows0, rows1, stage, sem0, sem1):
        wid = lax.axis_index("s") * NC + lax.axis_index("c")
        wbase = wid * BPW
        pltpu.sync_copy(x_hbm.at[pl.ds(wbase, BPW), :], idx_all)

        def fire(g, rows_ref, sem):
            for bb in range(CB):
                bag = g * CB + bb
                pltpu.async_copy(
                    tab_hbm.at[idx_all.at[bag, pl.ds(0, G0)]],
                    rows_ref.at[pl.ds(bb * L, G0), :], sem)
                pltpu.async_copy(
                    tab_hbm.at[idx_all.at[bag, pl.ds(G0, G1)]],
                    rows_ref.at[pl.ds(bb * L + G0, G1), :], sem)

        def drain(g, rows_ref, sem):
            for bb in range(CB):
                bag = g * CB + bb
                pltpu.make_async_copy(
                    tab_hbm.at[idx_all.at[bag, pl.ds(0, G0)]],
                    rows_ref.at[pl.ds(bb * L, G0), :], sem).wait()
                pltpu.make_async_copy(
                    tab_hbm.at[idx_all.at[bag, pl.ds(G0, G1)]],
                    rows_ref.at[pl.ds(bb * L + G0, G1), :], sem).wait()

        def accum(g, rows_ref):
            for bb in range(CB):
                base = bb * L
                zz = jnp.zeros((16,), jnp.float32)

                def body(i, carry, base=base, rows_ref=rows_ref):
                    a0, a1, a2, a3, a4, a5, a6, a7 = carry
                    r = base + 4 * i
                    a0 = a0 + rows_ref[r, pl.ds(0, 16)]
                    a1 = a1 + rows_ref[r, pl.ds(16, 16)]
                    a2 = a2 + rows_ref[r + 1, pl.ds(0, 16)]
                    a3 = a3 + rows_ref[r + 1, pl.ds(16, 16)]
                    a4 = a4 + rows_ref[r + 2, pl.ds(0, 16)]
                    a5 = a5 + rows_ref[r + 2, pl.ds(16, 16)]
                    a6 = a6 + rows_ref[r + 3, pl.ds(0, 16)]
                    a7 = a7 + rows_ref[r + 3, pl.ds(16, 16)]
                    return (a0, a1, a2, a3, a4, a5, a6, a7)

                a = lax.fori_loop(0, L // 4, body, (zz,) * 8)
                bag = g * CB + bb
                stage[bag, pl.ds(0, 16)] = (a[0] + a[2]) + (a[4] + a[6])
                stage[bag, pl.ds(16, 16)] = (a[1] + a[3]) + (a[5] + a[7])

        fire(0, rows0, sem0)

        @pl.loop(0, NCHUNK, step=2)
        def _(g):
            fire(g + 1, rows1, sem1)
            drain(g, rows0, sem0)
            accum(g, rows0)

            @pl.when(g + 2 < NCHUNK)
            def _():
                fire(g + 2, rows0, sem0)

            drain(g + 1, rows1, sem1)
            accum(g + 1, rows1)

        pltpu.sync_copy(stage, out_hbm.at[pl.ds(wbase, BPW), :])

    return k(x, table)


def _tc_head(x, summed, w, b2):
    C = w.shape[0]
    BT = 512

    def body(x_ref, s_ref, w_ref, b_ref, o_ref):
        cnt = jnp.sum((x_ref[...] != 0).astype(jnp.float32), axis=1, keepdims=True)
        denom = jnp.maximum(cnt, 1.0)
        acc = lax.dot_general(
            s_ref[...], w_ref[...], (((1,), (1,)), ((), ())),
            preferred_element_type=jnp.float32,
        )
        o_ref[...] = acc / denom + b_ref[...]

    return pl.pallas_call(
        body,
        grid=(B // BT,),
        in_specs=[
            pl.BlockSpec((BT, L), lambda i: (i, 0)),
            pl.BlockSpec((BT, D), lambda i: (i, 0)),
            pl.BlockSpec((C, D), lambda i: (0, 0)),
            pl.BlockSpec((1, C), lambda i: (0, 0)),
        ],
        out_specs=pl.BlockSpec((BT, C), lambda i: (i, 0)),
        out_shape=jax.ShapeDtypeStruct((B, C), jnp.float32),
    )(x, summed, w, b2)


def kernel(x, table, W, b):
    x = x.astype(jnp.int32)
    summed = _sc_bag_sums(x, table)
    return _tc_head(x, summed, W, b.reshape(1, -1))
